# initial kernel scaffold (unmeasured)
import jax
import jax.numpy as jnp
from jax import lax
from jax.experimental import pallas as pl
from jax.experimental.pallas import tpu as pltpu


def kernel(
    x,
):
    def body(*refs):
        pass

    out_shape = jax.ShapeDtypeStruct(..., jnp.float32)
    return pl.pallas_call(body, out_shape=out_shape)(...)


# baseline (device time: 29891 ns/iter reference)
import jax
import jax.numpy as jnp
from jax import lax
from jax.experimental import pallas as pl
from jax.experimental.pallas import tpu as pltpu

N_DEV = 32


def kernel(x):
    m, n = x.shape

    def body(x_ref, out_ref, send_buf, recv_buf, send_sem, recv_sem):
        my = lax.axis_index("i")

        y = x_ref[...]
        s = 1
        while s < m:
            ones = jnp.ones((s, n), y.dtype)
            y = y * jnp.concatenate([ones, y[: m - s, :]], axis=0)
            s *= 2
        last = y[m - 1 :, :]

        @pl.when(my > 0)
        def _():
            recv = pltpu.make_async_remote_copy(
                src_ref=send_buf,
                dst_ref=recv_buf,
                send_sem=send_sem,
                recv_sem=recv_sem,
                device_id=((my - 1) % N_DEV,),
                device_id_type=pl.DeviceIdType.MESH,
            )
            recv.wait_recv()

        @pl.when(my == 0)
        def _():
            recv_buf[...] = jnp.ones((1, n), y.dtype)

        prefix = recv_buf[...]
        send_buf[...] = prefix * last

        @pl.when(my < N_DEV - 1)
        def _():
            send = pltpu.make_async_remote_copy(
                src_ref=send_buf,
                dst_ref=recv_buf,
                send_sem=send_sem,
                recv_sem=recv_sem,
                device_id=(my + 1,),
                device_id_type=pl.DeviceIdType.MESH,
            )
            send.start()

        out_ref[...] = y * prefix

        @pl.when(my < N_DEV - 1)
        def _():
            done = pltpu.make_async_remote_copy(
                src_ref=send_buf,
                dst_ref=recv_buf,
                send_sem=send_sem,
                recv_sem=recv_sem,
                device_id=(my + 1,),
                device_id_type=pl.DeviceIdType.MESH,
            )
            done.wait_send()

    return pl.pallas_call(
        body,
        out_shape=jax.ShapeDtypeStruct((m, n), x.dtype),
        in_specs=[pl.BlockSpec(memory_space=pltpu.VMEM)],
        out_specs=pl.BlockSpec(memory_space=pltpu.VMEM),
        scratch_shapes=[
            pltpu.VMEM((1, n), x.dtype),
            pltpu.VMEM((1, n), x.dtype),
            pltpu.SemaphoreType.DMA,
            pltpu.SemaphoreType.DMA,
        ],
    )(x)


# device time: 28720 ns/iter; 1.0408x vs baseline; 1.0408x over previous
import jax
import jax.numpy as jnp
from jax import lax
from jax.experimental import pallas as pl
from jax.experimental.pallas import tpu as pltpu

N_DEV = 32
STEPS = (1, 2, 4, 8, 16)


def kernel(x):
    m, n = x.shape
    cdt = jnp.float32

    def body(x_ref, out_ref, send_bufs, recv_bufs, send_sems, recv_sems):
        my = lax.axis_index("i")

        y = x_ref[...].astype(cdt)
        s = 1
        while s < m:
            ones = jnp.ones((s, n), cdt)
            y = y * jnp.concatenate([ones, y[: m - s, :]], axis=0)
            s *= 2

        inc = y[m - 1 :, :]
        exc = jnp.ones((1, n), cdt)
        for t, s in enumerate(STEPS):

            @pl.when(my + s < N_DEV)
            def _(t=t, s=s, inc=inc):
                send_bufs[t] = inc
                rdma = pltpu.make_async_remote_copy(
                    src_ref=send_bufs.at[t],
                    dst_ref=recv_bufs.at[t],
                    send_sem=send_sems.at[t],
                    recv_sem=recv_sems.at[t],
                    device_id=(my + s,),
                    device_id_type=pl.DeviceIdType.MESH,
                )
                rdma.start()

            @pl.when(my < s)
            def _(t=t):
                recv_bufs[t] = jnp.ones((1, n), cdt)

            @pl.when(my >= s)
            def _(t=t, s=s):
                rdma = pltpu.make_async_remote_copy(
                    src_ref=send_bufs.at[t],
                    dst_ref=recv_bufs.at[t],
                    send_sem=send_sems.at[t],
                    recv_sem=recv_sems.at[t],
                    device_id=((my - s) % N_DEV,),
                    device_id_type=pl.DeviceIdType.MESH,
                )
                rdma.wait_recv()

            got = recv_bufs[t]
            exc = exc * got
            inc = inc * got

        out_ref[...] = (y * exc).astype(out_ref.dtype)

        for t, s in enumerate(STEPS):

            @pl.when(my + s < N_DEV)
            def _(t=t, s=s):
                rdma = pltpu.make_async_remote_copy(
                    src_ref=send_bufs.at[t],
                    dst_ref=recv_bufs.at[t],
                    send_sem=send_sems.at[t],
                    recv_sem=recv_sems.at[t],
                    device_id=(my + s,),
                    device_id_type=pl.DeviceIdType.MESH,
                )
                rdma.wait_send()

    k = len(STEPS)
    return pl.pallas_call(
        body,
        out_shape=jax.ShapeDtypeStruct((m, n), x.dtype),
        in_specs=[pl.BlockSpec(memory_space=pltpu.VMEM)],
        out_specs=pl.BlockSpec(memory_space=pltpu.VMEM),
        scratch_shapes=[
            pltpu.VMEM((k, 1, n), cdt),
            pltpu.VMEM((k, 1, n), cdt),
            pltpu.SemaphoreType.DMA((k,)),
            pltpu.SemaphoreType.DMA((k,)),
        ],
    )(x)
